# baseline (speedup 1.0000x reference)
import functools

import jax
import jax.numpy as jnp
from jax import lax
from jax.experimental import pallas as pl
from jax.experimental.pallas import tpu as pltpu
from jax.experimental.pallas import tpu_sc as plsc

E = 64
H = 1024
F = 256
T = 512
P = 2 * T
BLK = 16
NBLK = 128
PADROWS = NBLK * BLK
QCHUNK = 512
NQ = PADROWS // QCHUNK


def _router_meta_body(flat_ref, wg_ref, logits_ref, se_ref, pos0_ref, pos1_ref,
                      gt_ref, sw_ref, posr0_ref, posr1_ref, wr0_ref, wr1_ref):
    c = pl.program_id(0)

    @pl.when(c == 0)
    def _():
        flat = flat_ref[...]
        wg = wg_ref[...]
        logits = lax.dot_general(flat, wg, (((1,), (1,)), ((), ())),
                                 preferred_element_type=jnp.float32)
        logits_ref[...] = logits
        m = jnp.max(logits, axis=1, keepdims=True)
        ex = jnp.exp(logits - m)
        probs = ex / jnp.sum(ex, axis=1, keepdims=True)
        eiota = lax.broadcasted_iota(jnp.int32, (T, E), 1).astype(jnp.float32)
        m1 = jnp.max(probs, axis=1, keepdims=True)
        idx1 = jnp.min(jnp.where(probs == m1, eiota, float(E)), axis=1,
                       keepdims=True)
        oh1 = (eiota == idx1).astype(jnp.float32)
        probs2 = jnp.where(oh1 > 0.0, -1.0, probs)
        m2 = jnp.max(probs2, axis=1, keepdims=True)
        idx2 = jnp.min(jnp.where(probs2 == m2, eiota, float(E)), axis=1,
                       keepdims=True)
        oh2 = (eiota == idx2).astype(jnp.float32)
        denom = m1 + m2
        w1 = m1 / denom
        w2 = m2 / denom
        O = jnp.concatenate([oh1, oh2], axis=0)
        wv = jnp.concatenate([w1, w2], axis=0)
        counts = jnp.sum(O, axis=0, keepdims=True)
        nb = jnp.floor((counts + float(BLK - 1)) * (1.0 / BLK))
        er = lax.broadcasted_iota(jnp.int32, (E, E), 0).astype(jnp.float32)
        ec = lax.broadcasted_iota(jnp.int32, (E, E), 1).astype(jnp.float32)
        ustrict = (er < ec).astype(jnp.float32)
        nb8 = jnp.broadcast_to(nb, (8, E))
        blk8 = lax.dot_general(nb8, ustrict, (((1,), (0,)), ((), ())))
        blk_off = blk8[0:1]
        ends = blk_off + nb
        sio = lax.broadcasted_iota(jnp.int32, (NBLK, E), 0).astype(jnp.float32)
        done = (jnp.broadcast_to(ends, (NBLK, E)) <= sio).astype(jnp.float32)
        se = jnp.sum(done, axis=1, keepdims=True)
        se_ref[...] = jnp.minimum(se, float(E - 1)).astype(jnp.int32)
        pr = lax.broadcasted_iota(jnp.int32, (P, P), 0).astype(jnp.float32)
        pc = lax.broadcasted_iota(jnp.int32, (P, P), 1).astype(jnp.float32)
        lincl = (pr >= pc).astype(jnp.float32)
        csum = lax.dot_general(lincl, O, (((1,), (0,)), ((), ())))
        off = blk_off * float(BLK)
        pos_pe = O * (csum - 1.0 + jnp.broadcast_to(off, (P, E)))
        posall = jnp.sum(pos_pe, axis=1, keepdims=True)
        pos0_ref[...] = posall[:T].astype(jnp.int32)
        pos1_ref[...] = posall[T:].astype(jnp.int32)
        pio = lax.broadcasted_iota(jnp.int32, (P, T), 0)
        tio2 = lax.broadcasted_iota(jnp.int32, (P, T), 1)
        sel0 = (pio == tio2).astype(jnp.float32)
        sel1 = ((pio - T) == tio2).astype(jnp.float32)
        dnum = (((0,), (0,)), ((), ()))
        hi = lax.Precision.HIGHEST
        posr0_ref[...] = lax.dot_general(posall, sel0, dnum, precision=hi)
        posr1_ref[...] = lax.dot_general(posall, sel1, dnum, precision=hi)
        wr0_ref[...] = lax.dot_general(wv, sel0, dnum, precision=hi)
        wr1_ref[...] = lax.dot_general(wv, sel1, dnum, precision=hi)

    p0 = jnp.broadcast_to(posr0_ref[...], (QCHUNK, T))
    p1 = jnp.broadcast_to(posr1_ref[...], (QCHUNK, T))
    qbase = (c * QCHUNK).astype(jnp.float32)
    qio = lax.broadcasted_iota(jnp.int32, (QCHUNK, T), 0).astype(jnp.float32)
    qio = qio + qbase
    g0 = (qio == p0).astype(jnp.float32)
    g1 = (qio == p1).astype(jnp.float32)
    gt_ref[...] = g0 + g1
    w0 = jnp.broadcast_to(wr0_ref[...], (QCHUNK, T))
    w1 = jnp.broadcast_to(wr1_ref[...], (QCHUNK, T))
    sw_ref[...] = jnp.sum(g0 * w0 + g1 * w1, axis=1, keepdims=True)


def _run_router_meta(flat, wg):
    return pl.pallas_call(
        _router_meta_body,
        grid=(NQ,),
        in_specs=[
            pl.BlockSpec((T, H), lambda c: (0, 0)),
            pl.BlockSpec((E, H), lambda c: (0, 0)),
        ],
        out_specs=[
            pl.BlockSpec((T, E), lambda c: (0, 0)),
            pl.BlockSpec((NBLK, 1), lambda c: (0, 0)),
            pl.BlockSpec((T, 1), lambda c: (0, 0)),
            pl.BlockSpec((T, 1), lambda c: (0, 0)),
            pl.BlockSpec((QCHUNK, T), lambda c: (c, 0)),
            pl.BlockSpec((QCHUNK, 1), lambda c: (c, 0)),
        ],
        out_shape=[
            jax.ShapeDtypeStruct((T, E), jnp.float32),
            jax.ShapeDtypeStruct((NBLK, 1), jnp.int32),
            jax.ShapeDtypeStruct((T, 1), jnp.int32),
            jax.ShapeDtypeStruct((T, 1), jnp.int32),
            jax.ShapeDtypeStruct((PADROWS, T), jnp.float32),
            jax.ShapeDtypeStruct((PADROWS, 1), jnp.float32),
        ],
        scratch_shapes=[
            pltpu.VMEM((1, T), jnp.float32),
            pltpu.VMEM((1, T), jnp.float32),
            pltpu.VMEM((1, T), jnp.float32),
            pltpu.VMEM((1, T), jnp.float32),
        ],
    )(flat, wg)


_NCORE = 2
_NSUB = 16
_NLANE = 16
_NW = _NCORE * _NSUB
_GROWS = PADROWS // _NW
_CTOK = T // _NW


def _mlp_body(se_ref, gt_ref, flat_ref, gw_ref, uw_ref, dw_ref, sw_ref, y_ref):
    x = lax.dot_general(gt_ref[...], flat_ref[...], (((1,), (0,)), ((), ())),
                        preferred_element_type=jnp.float32)
    gw = gw_ref[0]
    uw = uw_ref[0]
    dw = dw_ref[0]
    g = lax.dot_general(x, gw, (((1,), (1,)), ((), ())),
                        preferred_element_type=jnp.float32)
    u = lax.dot_general(x, uw, (((1,), (1,)), ((), ())),
                        preferred_element_type=jnp.float32)
    act = (g / (1.0 + jnp.exp(-g))) * u
    y = lax.dot_general(act, dw, (((1,), (1,)), ((), ())),
                        preferred_element_type=jnp.float32)
    y_ref[...] = y * sw_ref[...]


def _run_mlp(se, gt, flat, gate_w, up_w, down_w, sw):
    grid_spec = pltpu.PrefetchScalarGridSpec(
        num_scalar_prefetch=1,
        grid=(NBLK,),
        in_specs=[
            pl.BlockSpec((BLK, T), lambda s, se: (s, 0)),
            pl.BlockSpec((T, H), lambda s, se: (0, 0)),
            pl.BlockSpec((1, F, H), lambda s, se: (se[s], 0, 0)),
            pl.BlockSpec((1, F, H), lambda s, se: (se[s], 0, 0)),
            pl.BlockSpec((1, H, F), lambda s, se: (se[s], 0, 0)),
            pl.BlockSpec((BLK, 1), lambda s, se: (s, 0)),
        ],
        out_specs=pl.BlockSpec((BLK, H), lambda s, se: (s, 0)),
    )
    return pl.pallas_call(
        _mlp_body,
        grid_spec=grid_spec,
        out_shape=jax.ShapeDtypeStruct((PADROWS, H), jnp.float32),
    )(se, gt, flat, gate_w, up_w, down_w, sw)


def _sc_combine_body(ys_hbm, pos0_hbm, pos1_hbm, out_hbm,
                     i0_v, i1_v, a_v, b_v, sem0, sem1):
    wid = lax.axis_index("s") * _NCORE + lax.axis_index("c")
    base = wid * _CTOK
    pltpu.sync_copy(pos0_hbm.at[pl.ds(base, _CTOK)], i0_v)
    pltpu.sync_copy(pos1_hbm.at[pl.ds(base, _CTOK)], i1_v)
    cp_a = pltpu.async_copy(ys_hbm.at[i0_v], a_v, sem0)
    cp_b = pltpu.async_copy(ys_hbm.at[i1_v], b_v, sem1)
    cp_a.wait()
    cp_b.wait()
    nlane = _NLANE
    for t in range(_CTOK):
        def add_chunk(j, _, t=t):
            s = j * nlane
            a_v[t, pl.ds(s, nlane)] = (a_v[t, pl.ds(s, nlane)]
                                       + b_v[t, pl.ds(s, nlane)])
            return 0
        lax.fori_loop(0, H // nlane, add_chunk, 0)
    pltpu.sync_copy(a_v, out_hbm.at[pl.ds(base, _CTOK)])


def _run_combine(ys, pos0, pos1):
    fn = functools.partial(
        pl.kernel,
        mesh=plsc.VectorSubcoreMesh(core_axis_name="c", subcore_axis_name="s"),
        out_type=jax.ShapeDtypeStruct((T, H), jnp.float32),
        scratch_types=[
            pltpu.VMEM((_CTOK,), jnp.int32),
            pltpu.VMEM((_CTOK,), jnp.int32),
            pltpu.VMEM((_CTOK, H), jnp.float32),
            pltpu.VMEM((_CTOK, H), jnp.float32),
            pltpu.SemaphoreType.DMA,
            pltpu.SemaphoreType.DMA,
        ],
    )(_sc_combine_body)
    return fn(ys, pos0, pos1)


def kernel(hidden_states, Wg, gate_w, up_w, down_w):
    orig_shape = hidden_states.shape
    flat = hidden_states.reshape(T, H)
    logits, se, pos0, pos1, gt, sw = _run_router_meta(flat, Wg)
    ys = _run_mlp(se.reshape(NBLK), gt, flat, gate_w, up_w, down_w, sw)
    out = _run_combine(ys, pos0.reshape(T), pos1.reshape(T))
    return out.reshape(orig_shape), logits


# speedup vs baseline: 1.2138x; 1.2138x over previous
import functools

import jax
import jax.numpy as jnp
from jax import lax
from jax.experimental import pallas as pl
from jax.experimental.pallas import tpu as pltpu
from jax.experimental.pallas import tpu_sc as plsc

E = 64
H = 1024
F = 256
T = 512
P = 2 * T
BLK = 32
NBLK = 96
PADROWS = NBLK * BLK
QCHUNK = 512
NQ = PADROWS // QCHUNK


def _router_meta_body(flat_ref, wg_ref, logits_ref, se_ref, pos0_ref, pos1_ref,
                      xs_ref, sw_ref, posr0_ref, posr1_ref, wr0_ref, wr1_ref):
    c = pl.program_id(0)

    @pl.when(c == 0)
    def _():
        flat = flat_ref[...]
        wg = wg_ref[...]
        logits = lax.dot_general(flat, wg, (((1,), (1,)), ((), ())),
                                 preferred_element_type=jnp.float32)
        logits_ref[...] = logits
        m = jnp.max(logits, axis=1, keepdims=True)
        ex = jnp.exp(logits - m)
        probs = ex / jnp.sum(ex, axis=1, keepdims=True)
        eiota = lax.broadcasted_iota(jnp.int32, (T, E), 1).astype(jnp.float32)
        m1 = jnp.max(probs, axis=1, keepdims=True)
        idx1 = jnp.min(jnp.where(probs == m1, eiota, float(E)), axis=1,
                       keepdims=True)
        oh1 = (eiota == idx1).astype(jnp.float32)
        probs2 = jnp.where(oh1 > 0.0, -1.0, probs)
        m2 = jnp.max(probs2, axis=1, keepdims=True)
        idx2 = jnp.min(jnp.where(probs2 == m2, eiota, float(E)), axis=1,
                       keepdims=True)
        oh2 = (eiota == idx2).astype(jnp.float32)
        denom = m1 + m2
        w1 = m1 / denom
        w2 = m2 / denom
        O = jnp.concatenate([oh1, oh2], axis=0)
        wv = jnp.concatenate([w1, w2], axis=0)
        counts = jnp.sum(O, axis=0, keepdims=True)
        nb = jnp.floor((counts + float(BLK - 1)) * (1.0 / BLK))
        er = lax.broadcasted_iota(jnp.int32, (E, E), 0).astype(jnp.float32)
        ec = lax.broadcasted_iota(jnp.int32, (E, E), 1).astype(jnp.float32)
        ustrict = (er < ec).astype(jnp.float32)
        nb8 = jnp.broadcast_to(nb, (8, E))
        blk8 = lax.dot_general(nb8, ustrict, (((1,), (0,)), ((), ())))
        blk_off = blk8[0:1]
        ends = blk_off + nb
        sio = lax.broadcasted_iota(jnp.int32, (NBLK, E), 0).astype(jnp.float32)
        done = (jnp.broadcast_to(ends, (NBLK, E)) <= sio).astype(jnp.float32)
        se = jnp.sum(done, axis=1, keepdims=True)
        se_ref[...] = jnp.minimum(se, float(E - 1)).astype(jnp.int32)
        pr = lax.broadcasted_iota(jnp.int32, (P, P), 0).astype(jnp.float32)
        pc = lax.broadcasted_iota(jnp.int32, (P, P), 1).astype(jnp.float32)
        lincl = (pr >= pc).astype(jnp.float32)
        csum = lax.dot_general(lincl, O, (((1,), (0,)), ((), ())))
        off = blk_off * float(BLK)
        pos_pe = O * (csum - 1.0 + jnp.broadcast_to(off, (P, E)))
        posall = jnp.sum(pos_pe, axis=1, keepdims=True)
        pos0_ref[...] = posall[:T].astype(jnp.int32)
        pos1_ref[...] = posall[T:].astype(jnp.int32)
        pio = lax.broadcasted_iota(jnp.int32, (P, T), 0)
        tio2 = lax.broadcasted_iota(jnp.int32, (P, T), 1)
        sel0 = (pio == tio2).astype(jnp.float32)
        sel1 = ((pio - T) == tio2).astype(jnp.float32)
        dnum = (((0,), (0,)), ((), ()))
        hi = lax.Precision.HIGHEST
        posr0_ref[...] = lax.dot_general(posall, sel0, dnum, precision=hi)
        posr1_ref[...] = lax.dot_general(posall, sel1, dnum, precision=hi)
        wr0_ref[...] = lax.dot_general(wv, sel0, dnum, precision=hi)
        wr1_ref[...] = lax.dot_general(wv, sel1, dnum, precision=hi)

    p0 = jnp.broadcast_to(posr0_ref[...], (QCHUNK, T))
    p1 = jnp.broadcast_to(posr1_ref[...], (QCHUNK, T))
    qbase = (c * QCHUNK).astype(jnp.float32)
    qio = lax.broadcasted_iota(jnp.int32, (QCHUNK, T), 0).astype(jnp.float32)
    qio = qio + qbase
    g0 = (qio == p0).astype(jnp.float32)
    g1 = (qio == p1).astype(jnp.float32)
    xs_ref[...] = lax.dot_general(g0 + g1, flat_ref[...],
                                  (((1,), (0,)), ((), ())),
                                  preferred_element_type=jnp.float32)
    w0 = jnp.broadcast_to(wr0_ref[...], (QCHUNK, T))
    w1 = jnp.broadcast_to(wr1_ref[...], (QCHUNK, T))
    sw_ref[...] = jnp.sum(g0 * w0 + g1 * w1, axis=1, keepdims=True)


def _run_router_meta(flat, wg):
    return pl.pallas_call(
        _router_meta_body,
        grid=(NQ,),
        in_specs=[
            pl.BlockSpec((T, H), lambda c: (0, 0)),
            pl.BlockSpec((E, H), lambda c: (0, 0)),
        ],
        out_specs=[
            pl.BlockSpec((T, E), lambda c: (0, 0)),
            pl.BlockSpec((NBLK, 1), lambda c: (0, 0)),
            pl.BlockSpec((T, 1), lambda c: (0, 0)),
            pl.BlockSpec((T, 1), lambda c: (0, 0)),
            pl.BlockSpec((QCHUNK, H), lambda c: (c, 0)),
            pl.BlockSpec((QCHUNK, 1), lambda c: (c, 0)),
        ],
        out_shape=[
            jax.ShapeDtypeStruct((T, E), jnp.float32),
            jax.ShapeDtypeStruct((NBLK, 1), jnp.int32),
            jax.ShapeDtypeStruct((T, 1), jnp.int32),
            jax.ShapeDtypeStruct((T, 1), jnp.int32),
            jax.ShapeDtypeStruct((PADROWS, H), jnp.float32),
            jax.ShapeDtypeStruct((PADROWS, 1), jnp.float32),
        ],
        scratch_shapes=[
            pltpu.VMEM((1, T), jnp.float32),
            pltpu.VMEM((1, T), jnp.float32),
            pltpu.VMEM((1, T), jnp.float32),
            pltpu.VMEM((1, T), jnp.float32),
        ],
    )(flat, wg)


_NCORE = 2
_NSUB = 16
_NLANE = 16
_NW = _NCORE * _NSUB
_GROWS = PADROWS // _NW
_CTOK = T // _NW


def _mlp_body(se_ref, xs_ref, gw_ref, uw_ref, dw_ref, sw_ref, y_ref):
    x = xs_ref[...]
    gw = gw_ref[0]
    uw = uw_ref[0]
    dw = dw_ref[0]
    g = lax.dot_general(x, gw, (((1,), (1,)), ((), ())),
                        preferred_element_type=jnp.float32)
    u = lax.dot_general(x, uw, (((1,), (1,)), ((), ())),
                        preferred_element_type=jnp.float32)
    act = (g / (1.0 + jnp.exp(-g))) * u
    y = lax.dot_general(act, dw, (((1,), (1,)), ((), ())),
                        preferred_element_type=jnp.float32)
    y_ref[...] = y * sw_ref[...]


def _run_mlp(se, xs, gate_w, up_w, down_w, sw):
    grid_spec = pltpu.PrefetchScalarGridSpec(
        num_scalar_prefetch=1,
        grid=(NBLK,),
        in_specs=[
            pl.BlockSpec((BLK, H), lambda s, se: (s, 0)),
            pl.BlockSpec((1, F, H), lambda s, se: (se[s], 0, 0)),
            pl.BlockSpec((1, F, H), lambda s, se: (se[s], 0, 0)),
            pl.BlockSpec((1, H, F), lambda s, se: (se[s], 0, 0)),
            pl.BlockSpec((BLK, 1), lambda s, se: (s, 0)),
        ],
        out_specs=pl.BlockSpec((BLK, H), lambda s, se: (s, 0)),
    )
    return pl.pallas_call(
        _mlp_body,
        grid_spec=grid_spec,
        out_shape=jax.ShapeDtypeStruct((PADROWS, H), jnp.float32),
    )(se, xs, gate_w, up_w, down_w, sw)


def _sc_combine_body(ys_hbm, pos0_hbm, pos1_hbm, out_hbm,
                     i0_v, i1_v, a_v, b_v, sem0, sem1):
    wid = lax.axis_index("s") * _NCORE + lax.axis_index("c")
    base = wid * _CTOK
    pltpu.sync_copy(pos0_hbm.at[pl.ds(base, _CTOK)], i0_v)
    pltpu.sync_copy(pos1_hbm.at[pl.ds(base, _CTOK)], i1_v)
    cp_a = pltpu.async_copy(ys_hbm.at[i0_v], a_v, sem0)
    cp_b = pltpu.async_copy(ys_hbm.at[i1_v], b_v, sem1)
    cp_a.wait()
    cp_b.wait()
    nlane = _NLANE
    for t in range(_CTOK):
        def add_chunk(j, _, t=t):
            s = j * nlane
            a_v[t, pl.ds(s, nlane)] = (a_v[t, pl.ds(s, nlane)]
                                       + b_v[t, pl.ds(s, nlane)])
            return 0
        lax.fori_loop(0, H // nlane, add_chunk, 0)
    pltpu.sync_copy(a_v, out_hbm.at[pl.ds(base, _CTOK)])


def _run_combine(ys, pos0, pos1):
    fn = functools.partial(
        pl.kernel,
        mesh=plsc.VectorSubcoreMesh(core_axis_name="c", subcore_axis_name="s"),
        out_type=jax.ShapeDtypeStruct((T, H), jnp.float32),
        scratch_types=[
            pltpu.VMEM((_CTOK,), jnp.int32),
            pltpu.VMEM((_CTOK,), jnp.int32),
            pltpu.VMEM((_CTOK, H), jnp.float32),
            pltpu.VMEM((_CTOK, H), jnp.float32),
            pltpu.SemaphoreType.DMA,
            pltpu.SemaphoreType.DMA,
        ],
    )(_sc_combine_body)
    return fn(ys, pos0, pos1)


def kernel(hidden_states, Wg, gate_w, up_w, down_w):
    orig_shape = hidden_states.shape
    flat = hidden_states.reshape(T, H)
    logits, se, pos0, pos1, xs, sw = _run_router_meta(flat, Wg)
    ys = _run_mlp(se.reshape(NBLK), xs, gate_w, up_w, down_w, sw)
    out = _run_combine(ys, pos0.reshape(T), pos1.reshape(T))
    return out.reshape(orig_shape), logits


# speedup vs baseline: 1.2212x; 1.0060x over previous
import functools

import jax
import jax.numpy as jnp
from jax import lax
from jax.experimental import pallas as pl
from jax.experimental.pallas import tpu as pltpu
from jax.experimental.pallas import tpu_sc as plsc

E = 64
H = 1024
F = 256
T = 512
P = 2 * T
BLK = 32
NBLK = 96
PADROWS = NBLK * BLK
QCHUNK = 512
NQ = PADROWS // QCHUNK


def _router_meta_body(flat_ref, wg_ref, logits_ref, se_ref, pos0_ref, pos1_ref,
                      xs_ref, sw_ref, posr0_ref, posr1_ref, wr0_ref, wr1_ref):
    c = pl.program_id(0)

    @pl.when(c == 0)
    def _():
        flat = flat_ref[...]
        wg = wg_ref[...]
        logits = lax.dot_general(flat, wg, (((1,), (1,)), ((), ())),
                                 preferred_element_type=jnp.float32)
        logits_ref[...] = logits
        m = jnp.max(logits, axis=1, keepdims=True)
        ex = jnp.exp(logits - m)
        probs = ex / jnp.sum(ex, axis=1, keepdims=True)
        eiota = lax.broadcasted_iota(jnp.int32, (T, E), 1).astype(jnp.float32)
        m1 = jnp.max(probs, axis=1, keepdims=True)
        idx1 = jnp.min(jnp.where(probs == m1, eiota, float(E)), axis=1,
                       keepdims=True)
        oh1 = (eiota == idx1).astype(jnp.float32)
        probs2 = jnp.where(oh1 > 0.0, -1.0, probs)
        m2 = jnp.max(probs2, axis=1, keepdims=True)
        idx2 = jnp.min(jnp.where(probs2 == m2, eiota, float(E)), axis=1,
                       keepdims=True)
        oh2 = (eiota == idx2).astype(jnp.float32)
        denom = m1 + m2
        w1 = m1 / denom
        w2 = m2 / denom
        O = jnp.concatenate([oh1, oh2], axis=0)
        wv = jnp.concatenate([w1, w2], axis=0)
        counts = jnp.sum(O, axis=0, keepdims=True)
        nb = jnp.floor((counts + float(BLK - 1)) * (1.0 / BLK))
        er = lax.broadcasted_iota(jnp.int32, (E, E), 0).astype(jnp.float32)
        ec = lax.broadcasted_iota(jnp.int32, (E, E), 1).astype(jnp.float32)
        ustrict = (er < ec).astype(jnp.float32)
        nb8 = jnp.broadcast_to(nb, (8, E))
        blk8 = lax.dot_general(nb8, ustrict, (((1,), (0,)), ((), ())))
        blk_off = blk8[0:1]
        ends = blk_off + nb
        sio = lax.broadcasted_iota(jnp.int32, (NBLK, E), 0).astype(jnp.float32)
        done = (jnp.broadcast_to(ends, (NBLK, E)) <= sio).astype(jnp.float32)
        se = jnp.sum(done, axis=1, keepdims=True)
        se_ref[...] = jnp.minimum(se, float(E - 1)).astype(jnp.int32)
        pr = lax.broadcasted_iota(jnp.int32, (P, P), 0).astype(jnp.float32)
        pc = lax.broadcasted_iota(jnp.int32, (P, P), 1).astype(jnp.float32)
        lincl = (pr >= pc).astype(jnp.float32)
        csum = lax.dot_general(lincl, O, (((1,), (0,)), ((), ())))
        off = blk_off * float(BLK)
        pos_pe = O * (csum - 1.0 + jnp.broadcast_to(off, (P, E)))
        posall = jnp.sum(pos_pe, axis=1, keepdims=True)
        pos0_ref[...] = posall[:T].astype(jnp.int32)
        pos1_ref[...] = posall[T:].astype(jnp.int32)
        pio = lax.broadcasted_iota(jnp.int32, (P, T), 0)
        tio2 = lax.broadcasted_iota(jnp.int32, (P, T), 1)
        sel0 = (pio == tio2).astype(jnp.float32)
        sel1 = ((pio - T) == tio2).astype(jnp.float32)
        dnum = (((0,), (0,)), ((), ()))
        hi = lax.Precision.HIGHEST
        posr0_ref[...] = lax.dot_general(posall, sel0, dnum, precision=hi)
        posr1_ref[...] = lax.dot_general(posall, sel1, dnum, precision=hi)
        wr0_ref[...] = lax.dot_general(wv, sel0, dnum, precision=hi)
        wr1_ref[...] = lax.dot_general(wv, sel1, dnum, precision=hi)

    p0 = jnp.broadcast_to(posr0_ref[...], (QCHUNK, T))
    p1 = jnp.broadcast_to(posr1_ref[...], (QCHUNK, T))
    qbase = (c * QCHUNK).astype(jnp.float32)
    qio = lax.broadcasted_iota(jnp.int32, (QCHUNK, T), 0).astype(jnp.float32)
    qio = qio + qbase
    g0 = (qio == p0).astype(jnp.float32)
    g1 = (qio == p1).astype(jnp.float32)
    xs_ref[...] = lax.dot_general(g0 + g1, flat_ref[...],
                                  (((1,), (0,)), ((), ())),
                                  preferred_element_type=jnp.float32)
    w0 = jnp.broadcast_to(wr0_ref[...], (QCHUNK, T))
    w1 = jnp.broadcast_to(wr1_ref[...], (QCHUNK, T))
    sw_ref[...] = jnp.sum(g0 * w0 + g1 * w1, axis=1, keepdims=True)


def _run_router_meta(flat, wg):
    return pl.pallas_call(
        _router_meta_body,
        grid=(NQ,),
        in_specs=[
            pl.BlockSpec((T, H), lambda c: (0, 0)),
            pl.BlockSpec((E, H), lambda c: (0, 0)),
        ],
        out_specs=[
            pl.BlockSpec((T, E), lambda c: (0, 0)),
            pl.BlockSpec((NBLK, 1), lambda c: (0, 0)),
            pl.BlockSpec((T, 1), lambda c: (0, 0)),
            pl.BlockSpec((T, 1), lambda c: (0, 0)),
            pl.BlockSpec((QCHUNK, H), lambda c: (c, 0)),
            pl.BlockSpec((QCHUNK, 1), lambda c: (c, 0)),
        ],
        out_shape=[
            jax.ShapeDtypeStruct((T, E), jnp.float32),
            jax.ShapeDtypeStruct((NBLK, 1), jnp.int32),
            jax.ShapeDtypeStruct((T, 1), jnp.int32),
            jax.ShapeDtypeStruct((T, 1), jnp.int32),
            jax.ShapeDtypeStruct((PADROWS, H), jnp.float32),
            jax.ShapeDtypeStruct((PADROWS, 1), jnp.float32),
        ],
        scratch_shapes=[
            pltpu.VMEM((1, T), jnp.float32),
            pltpu.VMEM((1, T), jnp.float32),
            pltpu.VMEM((1, T), jnp.float32),
            pltpu.VMEM((1, T), jnp.float32),
        ],
    )(flat, wg)


_NCORE = 2
_NSUB = 16
_NLANE = 16
_NW = _NCORE * _NSUB
_GROWS = PADROWS // _NW
_CTOK = T // _NW


def _mlp_body(se_ref, xs_ref, gw_ref, uw_ref, dw_ref, sw_ref, y_ref):
    x = xs_ref[...]
    gw = gw_ref[0]
    uw = uw_ref[0]
    dw = dw_ref[0]
    g = lax.dot_general(x, gw, (((1,), (1,)), ((), ())),
                        preferred_element_type=jnp.float32)
    u = lax.dot_general(x, uw, (((1,), (1,)), ((), ())),
                        preferred_element_type=jnp.float32)
    act = (g / (1.0 + jnp.exp(-g))) * u * sw_ref[...]
    y_ref[...] = lax.dot_general(act, dw, (((1,), (1,)), ((), ())),
                                 preferred_element_type=jnp.float32)


def _run_mlp(se, xs, gate_w, up_w, down_w, sw):
    grid_spec = pltpu.PrefetchScalarGridSpec(
        num_scalar_prefetch=1,
        grid=(NBLK,),
        in_specs=[
            pl.BlockSpec((BLK, H), lambda s, se: (s, 0)),
            pl.BlockSpec((1, F, H), lambda s, se: (se[s], 0, 0)),
            pl.BlockSpec((1, F, H), lambda s, se: (se[s], 0, 0)),
            pl.BlockSpec((1, H, F), lambda s, se: (se[s], 0, 0)),
            pl.BlockSpec((BLK, 1), lambda s, se: (s, 0)),
        ],
        out_specs=pl.BlockSpec((BLK, H), lambda s, se: (s, 0)),
    )
    return pl.pallas_call(
        _mlp_body,
        grid_spec=grid_spec,
        out_shape=jax.ShapeDtypeStruct((PADROWS, H), jnp.float32),
    )(se, xs, gate_w, up_w, down_w, sw)


def _sc_combine_body(ys_hbm, pos0_hbm, pos1_hbm, out_hbm,
                     i0_v, i1_v, a_v, b_v, sem0, sem1):
    wid = lax.axis_index("s") * _NCORE + lax.axis_index("c")
    base = wid * _CTOK
    pltpu.sync_copy(pos0_hbm.at[pl.ds(base, _CTOK)], i0_v)
    pltpu.sync_copy(pos1_hbm.at[pl.ds(base, _CTOK)], i1_v)
    cp_a = pltpu.async_copy(ys_hbm.at[i0_v], a_v, sem0)
    cp_b = pltpu.async_copy(ys_hbm.at[i1_v], b_v, sem1)
    cp_a.wait()
    cp_b.wait()
    nlane = _NLANE
    for t in range(_CTOK):
        def add_chunk(j, _, t=t):
            s = j * nlane
            a_v[t, pl.ds(s, nlane)] = (a_v[t, pl.ds(s, nlane)]
                                       + b_v[t, pl.ds(s, nlane)])
            return 0
        lax.fori_loop(0, H // nlane, add_chunk, 0)
    pltpu.sync_copy(a_v, out_hbm.at[pl.ds(base, _CTOK)])


def _run_combine(ys, pos0, pos1):
    fn = functools.partial(
        pl.kernel,
        mesh=plsc.VectorSubcoreMesh(core_axis_name="c", subcore_axis_name="s"),
        out_type=jax.ShapeDtypeStruct((T, H), jnp.float32),
        scratch_types=[
            pltpu.VMEM((_CTOK,), jnp.int32),
            pltpu.VMEM((_CTOK,), jnp.int32),
            pltpu.VMEM((_CTOK, H), jnp.float32),
            pltpu.VMEM((_CTOK, H), jnp.float32),
            pltpu.SemaphoreType.DMA,
            pltpu.SemaphoreType.DMA,
        ],
    )(_sc_combine_body)
    return fn(ys, pos0, pos1)


def kernel(hidden_states, Wg, gate_w, up_w, down_w):
    orig_shape = hidden_states.shape
    flat = hidden_states.reshape(T, H)
    logits, se, pos0, pos1, xs, sw = _run_router_meta(flat, Wg)
    ys = _run_mlp(se.reshape(NBLK), xs, gate_w, up_w, down_w, sw)
    out = _run_combine(ys, pos0.reshape(T), pos1.reshape(T))
    return out.reshape(orig_shape), logits
